# baseline (speedup 1.0000x reference)
import functools

import jax
import jax.numpy as jnp
from jax import lax
from jax.experimental import pallas as pl
from jax.experimental.pallas import tpu as pltpu
from jax.experimental.pallas import tpu_sc as plsc

N = 10000
E = 320000
EMB = 128

NC = 2
NS = 16
NW = NC * NS
EPW = E // NW
CH = 40
NCHUNK = EPW // CH
NPAD = 10240
RPS = NPAD // NS

_SC_PARAMS = pltpu.CompilerParams(needs_layout_passes=False)


@functools.cache
def _mesh():
    return plsc.VectorSubcoreMesh(
        core_axis_name="c", subcore_axis_name="s",
        num_cores=NC, num_subcores=NS)


def _zero_vmem(ref, rows, lanes):
    z = jnp.zeros((16,), jnp.float32)
    per_row = lanes // 16

    def body(i, _):
        ref[i // per_row, pl.ds((i % per_row) * 16, 16)] = z
        return 0

    lax.fori_loop(0, rows * per_row, body, 0)


NB = 5
NGRP = NCHUNK // NB
IDP = 10
ZR = 40


@functools.cache
def _build_sc_edge():
    return functools.partial(
        pl.kernel,
        out_type=jax.ShapeDtypeStruct((2 * NPAD, EMB), jnp.float32),
        mesh=_mesh(),
        compiler_params=_SC_PARAMS,
        scratch_types=[
            pltpu.VMEM((IDP, 2, CH), jnp.int32),
            pltpu.VMEM((NB, CH, EMB), jnp.float32),
            pltpu.VMEM((ZR, EMB), jnp.float32),
            pltpu.VMEM_SHARED((NPAD, EMB), jnp.float32),
            [pltpu.SemaphoreType.DMA] * IDP,
            [pltpu.SemaphoreType.DMA] * NB,
            [pltpu.SemaphoreType.DMA] * NB,
        ],
    )(_sc_edge_body)


def _sc_edge_body(hxe2_hbm, cc3_hbm, out_hbm,
                  ibuf, rows_v, zbuf, aggr_sh, isems, gsems, ssems):
    c = lax.axis_index("c")
    s = lax.axis_index("s")
    w = c * NS + s

    def start_idx(i, q):
        pltpu.async_copy(cc3_hbm.at[w, i], ibuf.at[q], isems[q])

    def wait_idx(i, q):
        pltpu.make_async_copy(cc3_hbm.at[w, i], ibuf.at[q], isems[q]).wait()

    def start_gather(q, b):
        pltpu.async_copy(hxe2_hbm.at[ibuf.at[q, 0]], rows_v.at[b], gsems[b])

    def wait_gather(q, b):
        pltpu.make_async_copy(
            hxe2_hbm.at[ibuf.at[q, 0]], rows_v.at[b], gsems[b]).wait()

    def start_scatter(q, b):
        pltpu.async_copy(
            rows_v.at[b], aggr_sh.at[ibuf.at[q, 1]], ssems[b], add=True)

    def wait_scatter(q, b):
        pltpu.make_async_copy(
            rows_v.at[b], aggr_sh.at[ibuf.at[q, 1]], ssems[b]).wait()

    for q in range(IDP):
        start_idx(q, q)
    _zero_vmem(zbuf, ZR, EMB)
    for k in range(RPS // ZR):
        pltpu.sync_copy(zbuf, aggr_sh.at[pl.ds(s * RPS + k * ZR, ZR)])
    plsc.subcore_barrier()
    for b in range(NB):
        wait_idx(b, b)
        start_gather(b, b)

    def group(g2, _):
        for gg in range(2):
            for b in range(NB):
                i = g2 * IDP + gg * NB + b
                q = gg * NB + b
                wait_gather(q, b)
                start_scatter(q, b)

                @pl.when(i + NB < NCHUNK)
                def _():
                    wait_scatter(q, b)

                    @pl.when(i + IDP < NCHUNK)
                    def _():
                        start_idx(i + IDP, q)

                    qn = (gg * NB + b + NB) % IDP
                    wait_idx(i + NB, qn)
                    start_gather(qn, b)

        return 0

    lax.fori_loop(0, NGRP // 2, group, 0)
    for b in range(NB):
        wait_scatter(NB + b, b)
    plsc.subcore_barrier()
    pltpu.sync_copy(aggr_sh.at[pl.ds(s * RPS, RPS)],
                    out_hbm.at[pl.ds(c * NPAD + s * RPS, RPS)])


@functools.cache
def _build_sc_deg():
    return functools.partial(
        pl.kernel,
        out_type=jax.ShapeDtypeStruct((2 * NPAD, EMB), jnp.float32),
        mesh=_mesh(),
        compiler_params=_SC_PARAMS,
        scratch_types=[
            pltpu.VMEM((IDP, CH), jnp.int32),
            pltpu.VMEM((CH, EMB), jnp.float32),
            pltpu.VMEM((ZR, EMB), jnp.float32),
            pltpu.VMEM_SHARED((NPAD, EMB), jnp.float32),
            [pltpu.SemaphoreType.DMA] * IDP,
            [pltpu.SemaphoreType.DMA] * NB,
        ],
    )(_sc_deg_body)


def _sc_deg_body(row3_hbm, out_hbm, ibuf, ones_v, zbuf, acc_sh,
                 isems, ssems):
    c = lax.axis_index("c")
    s = lax.axis_index("s")
    w = c * NS + s

    def start_idx(i, q):
        pltpu.async_copy(row3_hbm.at[w, i], ibuf.at[q], isems[q])

    def wait_idx(i, q):
        pltpu.make_async_copy(row3_hbm.at[w, i], ibuf.at[q], isems[q]).wait()

    def start_scatter(q, b):
        pltpu.async_copy(ones_v, acc_sh.at[ibuf.at[q]], ssems[b], add=True)

    def wait_scatter(q, b):
        pltpu.make_async_copy(ones_v, acc_sh.at[ibuf.at[q]], ssems[b]).wait()

    for q in range(IDP):
        start_idx(q, q)

    one = jnp.full((16,), 1.0, jnp.float32)

    def fill_ones(i, _):
        ones_v[i // 8, pl.ds((i % 8) * 16, 16)] = one
        return 0

    lax.fori_loop(0, CH * 8, fill_ones, 0)
    _zero_vmem(zbuf, ZR, EMB)
    for k in range(RPS // ZR):
        pltpu.sync_copy(zbuf, acc_sh.at[pl.ds(s * RPS + k * ZR, ZR)])
    plsc.subcore_barrier()

    def group(g2, _):
        for gg in range(2):
            for b in range(NB):
                i = g2 * IDP + gg * NB + b
                q = gg * NB + b
                qp = (q + NB) % IDP

                @pl.when(i >= NB)
                def _():
                    wait_scatter(qp, b)

                    @pl.when(i + NB < NCHUNK)
                    def _():
                        start_idx(i + NB, qp)

                wait_idx(i, q)
                start_scatter(q, b)

        return 0

    lax.fori_loop(0, NGRP // 2, group, 0)
    for b in range(NB):
        wait_scatter(NB + b, b)
    plsc.subcore_barrier()
    pltpu.sync_copy(acc_sh.at[pl.ds(s * RPS, RPS)],
                    out_hbm.at[pl.ds(c * NPAD + s * RPS, RPS)])


FCH = 2000


@functools.cache
def _build_sc_final():
    return functools.partial(
        pl.kernel,
        out_type=jax.ShapeDtypeStruct((E,), jnp.float32),
        mesh=_mesh(),
        compiler_params=_SC_PARAMS,
        scratch_types=[
            pltpu.VMEM((NPAD,), jnp.float32),
            pltpu.VMEM((NPAD,), jnp.float32),
            pltpu.VMEM((FCH,), jnp.int32),
            pltpu.VMEM((FCH,), jnp.int32),
            pltpu.VMEM((FCH,), jnp.float32),
        ],
    )(_sc_final_body)


def _sc_final_body(s1_hbm, s2_hbm, row_hbm, col_hbm, out_hbm,
                   s1_v, s2_v, row_v, col_v, out_v):
    c = lax.axis_index("c")
    s = lax.axis_index("s")
    base = c * (E // NC) + s * EPW

    pltpu.sync_copy(s1_hbm, s1_v)
    pltpu.sync_copy(s2_hbm, s2_v)

    def chunk(i, _):
        off = base + i * FCH
        pltpu.sync_copy(row_hbm.at[pl.ds(off, FCH)], row_v)
        pltpu.sync_copy(col_hbm.at[pl.ds(off, FCH)], col_v)

        def group(j, _):
            r = row_v[pl.ds(j * 16, 16)]
            cc = col_v[pl.ds(j * 16, 16)]
            v = plsc.load_gather(s1_v, [r]) + plsc.load_gather(s2_v, [cc])
            out_v[pl.ds(j * 16, 16)] = v
            return 0

        lax.fori_loop(0, FCH // 16, group, 0)
        pltpu.sync_copy(out_v, out_hbm.at[pl.ds(off, FCH)])
        return 0

    lax.fori_loop(0, EPW // FCH, chunk, 0)


BLK = 640
G = NPAD // BLK


def _tc_first_body(x_ref, at_ref, d0_ref, d1_ref, w_ref, b_ref, bond_ref,
                   h_ref, dis_ref, dinv_ref, hx_ref, hxe2_ref):
    i = pl.program_id(0)
    rows = lax.broadcasted_iota(jnp.int32, (BLK, 1), 0) + i * BLK
    mask = (rows < N).astype(jnp.float32)
    oh = (x_ref[...] == lax.broadcasted_iota(jnp.int32, (1, 16), 1)
          ).astype(jnp.float32)
    h = jnp.dot(oh, at_ref[...], preferred_element_type=jnp.float32)
    h_ref[...] = h
    deg = d0_ref[0, :, 0:1] + d1_ref[0, :, 0:1] + 1.0
    dis = lax.rsqrt(deg) * mask
    dis_ref[...] = dis
    dinv_ref[...] = mask / deg
    hx = jnp.dot(h, w_ref[...], preferred_element_type=jnp.float32) + b_ref[...]
    hx_ref[...] = hx
    for a in range(4):
        hxe2_ref[a] = dis * jnp.maximum(hx + bond_ref[a:a + 1, :], 0.0)


_tc_first = pl.pallas_call(
    _tc_first_body,
    grid=(G,),
    in_specs=[
        pl.BlockSpec((BLK, 1), lambda i: (i, 0)),
        pl.BlockSpec((16, EMB), lambda i: (0, 0)),
        pl.BlockSpec((1, BLK, EMB), lambda i: (0, i, 0)),
        pl.BlockSpec((1, BLK, EMB), lambda i: (1, i, 0)),
        pl.BlockSpec((EMB, EMB), lambda i: (0, 0)),
        pl.BlockSpec((1, EMB), lambda i: (0, 0)),
        pl.BlockSpec((4, EMB), lambda i: (0, 0)),
    ],
    out_specs=[
        pl.BlockSpec((BLK, EMB), lambda i: (i, 0)),
        pl.BlockSpec((BLK, 1), lambda i: (i, 0)),
        pl.BlockSpec((BLK, 1), lambda i: (i, 0)),
        pl.BlockSpec((BLK, EMB), lambda i: (i, 0)),
        pl.BlockSpec((4, BLK, EMB), lambda i: (0, i, 0)),
    ],
    out_shape=(
        jax.ShapeDtypeStruct((NPAD, EMB), jnp.float32),
        jax.ShapeDtypeStruct((NPAD, 1), jnp.float32),
        jax.ShapeDtypeStruct((NPAD, 1), jnp.float32),
        jax.ShapeDtypeStruct((NPAD, EMB), jnp.float32),
        jax.ShapeDtypeStruct((4, NPAD, EMB), jnp.float32),
    ),
)


def _tc_comb_body(row_ref, attr_ref, comb_ref):
    comb_ref[...] = attr_ref[...] * NPAD + row_ref[...]


_tc_comb = pl.pallas_call(
    _tc_comb_body,
    out_shape=jax.ShapeDtypeStruct((E // EMB, EMB), jnp.int32),
)


def _tc_stats_body(p0_ref, p1_ref, hx_ref, dis_ref, dinv_ref, root_ref,
                   conv_ref, sums_ref, acc):
    i = pl.program_id(0)
    conv = (dis_ref[...] * (p0_ref[0] + p1_ref[0])
            + jnp.maximum(hx_ref[...] + root_ref[...], 0.0) * dinv_ref[...])
    conv_ref[...] = conv

    @pl.when(i == 0)
    def _():
        acc[...] = jnp.zeros((2, EMB), jnp.float32)

    acc[0:1, :] += jnp.sum(conv, axis=0, keepdims=True)
    acc[1:2, :] += jnp.sum(conv * conv, axis=0, keepdims=True)

    @pl.when(i == G - 1)
    def _():
        sums_ref[...] = acc[...]


_tc_stats = pl.pallas_call(
    _tc_stats_body,
    grid=(G,),
    in_specs=[
        pl.BlockSpec((1, BLK, EMB), lambda i: (0, i, 0)),
        pl.BlockSpec((1, BLK, EMB), lambda i: (1, i, 0)),
        pl.BlockSpec((BLK, EMB), lambda i: (i, 0)),
        pl.BlockSpec((BLK, 1), lambda i: (i, 0)),
        pl.BlockSpec((BLK, 1), lambda i: (i, 0)),
        pl.BlockSpec((1, EMB), lambda i: (0, 0)),
    ],
    out_specs=[
        pl.BlockSpec((BLK, EMB), lambda i: (i, 0)),
        pl.BlockSpec((2, EMB), lambda i: (0, 0)),
    ],
    out_shape=(
        jax.ShapeDtypeStruct((NPAD, EMB), jnp.float32),
        jax.ShapeDtypeStruct((2, EMB), jnp.float32),
    ),
    scratch_shapes=[pltpu.VMEM((2, EMB), jnp.float32)],
)


def _bn_next(conv_ref, h_ref, sums_ref, gamma_ref, beta_ref):
    mean = sums_ref[0:1, :] * (1.0 / N)
    var = sums_ref[1:2, :] * (1.0 / N) - mean * mean
    bn = ((conv_ref[...] - mean) * lax.rsqrt(var + 1e-5) * gamma_ref[...]
          + beta_ref[...])
    return jnp.maximum(bn, 0.0) + h_ref[...]


def _tc_apply_body(conv_ref, h_ref, sums_ref, gamma_ref, beta_ref,
                   w_ref, b_ref, bond_ref, dis_ref,
                   hn_ref, hx_ref, hxe2_ref):
    hn = _bn_next(conv_ref, h_ref, sums_ref, gamma_ref, beta_ref)
    hn_ref[...] = hn
    hx = jnp.dot(hn, w_ref[...], preferred_element_type=jnp.float32) + b_ref[...]
    hx_ref[...] = hx
    dis = dis_ref[...]
    for a in range(4):
        hxe2_ref[a] = dis * jnp.maximum(hx + bond_ref[a:a + 1, :], 0.0)


_tc_apply = pl.pallas_call(
    _tc_apply_body,
    grid=(G,),
    in_specs=[
        pl.BlockSpec((BLK, EMB), lambda i: (i, 0)),
        pl.BlockSpec((BLK, EMB), lambda i: (i, 0)),
        pl.BlockSpec((2, EMB), lambda i: (0, 0)),
        pl.BlockSpec((1, EMB), lambda i: (0, 0)),
        pl.BlockSpec((1, EMB), lambda i: (0, 0)),
        pl.BlockSpec((EMB, EMB), lambda i: (0, 0)),
        pl.BlockSpec((1, EMB), lambda i: (0, 0)),
        pl.BlockSpec((4, EMB), lambda i: (0, 0)),
        pl.BlockSpec((BLK, 1), lambda i: (i, 0)),
    ],
    out_specs=[
        pl.BlockSpec((BLK, EMB), lambda i: (i, 0)),
        pl.BlockSpec((BLK, EMB), lambda i: (i, 0)),
        pl.BlockSpec((4, BLK, EMB), lambda i: (0, i, 0)),
    ],
    out_shape=(
        jax.ShapeDtypeStruct((NPAD, EMB), jnp.float32),
        jax.ShapeDtypeStruct((NPAD, EMB), jnp.float32),
        jax.ShapeDtypeStruct((4, NPAD, EMB), jnp.float32),
    ),
)


def _tc_last_body(conv_ref, h_ref, sums_ref, gamma_ref, beta_ref,
                  w1_ref, w2_ref, be_ref, s1_ref, s2_ref):
    hn = _bn_next(conv_ref, h_ref, sums_ref, gamma_ref, beta_ref)
    s1_ref[...] = jnp.dot(hn, w1_ref[...],
                          preferred_element_type=jnp.float32) + be_ref[...]
    s2_ref[...] = jnp.dot(hn, w2_ref[...], preferred_element_type=jnp.float32)


_tc_last = pl.pallas_call(
    _tc_last_body,
    grid=(G,),
    in_specs=[
        pl.BlockSpec((BLK, EMB), lambda i: (i, 0)),
        pl.BlockSpec((BLK, EMB), lambda i: (i, 0)),
        pl.BlockSpec((2, EMB), lambda i: (0, 0)),
        pl.BlockSpec((1, EMB), lambda i: (0, 0)),
        pl.BlockSpec((1, EMB), lambda i: (0, 0)),
        pl.BlockSpec((EMB, 1), lambda i: (0, 0)),
        pl.BlockSpec((EMB, 1), lambda i: (0, 0)),
        pl.BlockSpec((1, 1), lambda i: (0, 0)),
    ],
    out_specs=[
        pl.BlockSpec((BLK, 1), lambda i: (i, 0)),
        pl.BlockSpec((BLK, 1), lambda i: (i, 0)),
    ],
    out_shape=(
        jax.ShapeDtypeStruct((NPAD, 1), jnp.float32),
        jax.ShapeDtypeStruct((NPAD, 1), jnp.float32),
    ),
)


def kernel(x, edge_index, edge_attr, atom_table, W_lin, b_lin, root_emb,
           bond_table, bn_gamma, bn_beta, W_ep, b_ep):
    xp = jnp.pad(x.astype(jnp.int32), (0, NPAD - N)).reshape(NPAD, 1)
    row = edge_index[0].astype(jnp.int32)
    col = edge_index[1].astype(jnp.int32)
    attr = edge_attr.astype(jnp.int32)
    row3 = row.reshape(NW, NCHUNK, CH)
    col3 = col.reshape(NW, NCHUNK, CH)

    degp = _build_sc_deg()(row3)
    degp_r = degp.reshape(2, NPAD, EMB)
    h, dis, dinv, hx, hxe2 = _tc_first(
        xp, atom_table, degp_r, degp_r, W_lin[0],
        b_lin[0].reshape(1, EMB), bond_table[0])
    comb2 = _tc_comb(row.reshape(E // EMB, EMB), attr.reshape(E // EMB, EMB))
    cc3 = jnp.stack([comb2.reshape(NW, NCHUNK, CH), col3], axis=2)

    for i in range(3):
        parts = _build_sc_edge()(hxe2.reshape(4 * NPAD, EMB), cc3)
        parts_r = parts.reshape(2, NPAD, EMB)
        conv, sums = _tc_stats(parts_r, parts_r, hx, dis, dinv, root_emb[i])
        if i < 2:
            h, hx, hxe2 = _tc_apply(
                conv, h, sums, bn_gamma[i].reshape(1, EMB),
                bn_beta[i].reshape(1, EMB), W_lin[i + 1],
                b_lin[i + 1].reshape(1, EMB), bond_table[i + 1], dis)
        else:
            s1, s2 = _tc_last(
                conv, h, sums, bn_gamma[i].reshape(1, EMB),
                bn_beta[i].reshape(1, EMB), W_ep[:EMB], W_ep[EMB:],
                b_ep.reshape(1, 1))

    out = _build_sc_final()(s1.reshape(NPAD), s2.reshape(NPAD), row, col)
    return out.reshape(E, 1)


# speedup vs baseline: 1.0727x; 1.0727x over previous
import functools

import jax
import jax.numpy as jnp
from jax import lax
from jax.experimental import pallas as pl
from jax.experimental.pallas import tpu as pltpu
from jax.experimental.pallas import tpu_sc as plsc

N = 10000
E = 320000
EMB = 128

NC = 2
NS = 16
NW = NC * NS
EPW = E // NW
CH = 40
NCHUNK = EPW // CH
NPAD = 10240
RPS = NPAD // NS

_SC_PARAMS = pltpu.CompilerParams(needs_layout_passes=False)


@functools.cache
def _mesh():
    return plsc.VectorSubcoreMesh(
        core_axis_name="c", subcore_axis_name="s",
        num_cores=NC, num_subcores=NS)


def _zero_vmem(ref, rows, lanes):
    z = jnp.zeros((16,), jnp.float32)
    per_row = lanes // 16

    def body(i, _):
        ref[i // per_row, pl.ds((i % per_row) * 16, 16)] = z
        return 0

    lax.fori_loop(0, rows * per_row, body, 0)


NB = 5
NGRP = NCHUNK // NB
IDP = 10
ZR = 40


@functools.cache
def _build_sc_edge():
    return functools.partial(
        pl.kernel,
        out_type=jax.ShapeDtypeStruct((2 * NPAD, EMB), jnp.float32),
        mesh=_mesh(),
        compiler_params=_SC_PARAMS,
        scratch_types=[
            pltpu.VMEM((IDP, 2, CH), jnp.int32),
            pltpu.VMEM((NB, CH, EMB), jnp.float32),
            pltpu.VMEM((ZR, EMB), jnp.float32),
            pltpu.VMEM_SHARED((NPAD, EMB), jnp.float32),
            [pltpu.SemaphoreType.DMA] * IDP,
            [pltpu.SemaphoreType.DMA] * NB,
            [pltpu.SemaphoreType.DMA] * NB,
        ],
    )(_sc_edge_body)


def _sc_edge_body(hxe2_hbm, cc3_hbm, out_hbm,
                  ibuf, rows_v, zbuf, aggr_sh, isems, gsems, ssems):
    c = lax.axis_index("c")
    s = lax.axis_index("s")
    w = c * NS + s

    def start_idx(i, q):
        pltpu.async_copy(cc3_hbm.at[w, i], ibuf.at[q], isems[q])

    def wait_idx(i, q):
        pltpu.make_async_copy(cc3_hbm.at[w, i], ibuf.at[q], isems[q]).wait()

    def start_gather(q, b):
        pltpu.async_copy(hxe2_hbm.at[ibuf.at[q, 0]], rows_v.at[b], gsems[b])

    def wait_gather(q, b):
        pltpu.make_async_copy(
            hxe2_hbm.at[ibuf.at[q, 0]], rows_v.at[b], gsems[b]).wait()

    def start_scatter(q, b):
        pltpu.async_copy(
            rows_v.at[b], aggr_sh.at[ibuf.at[q, 1]], ssems[b], add=True)

    def wait_scatter(q, b):
        pltpu.make_async_copy(
            rows_v.at[b], aggr_sh.at[ibuf.at[q, 1]], ssems[b]).wait()

    for q in range(IDP):
        start_idx(q, q)
    _zero_vmem(zbuf, ZR, EMB)
    for k in range(RPS // ZR):
        pltpu.sync_copy(zbuf, aggr_sh.at[pl.ds(s * RPS + k * ZR, ZR)])
    plsc.subcore_barrier()
    for b in range(NB):
        wait_idx(b, b)
        start_gather(b, b)

    def group(g2, _):
        for gg in range(2):
            for b in range(NB):
                i = g2 * IDP + gg * NB + b
                q = gg * NB + b
                wait_gather(q, b)
                start_scatter(q, b)

                @pl.when(i + NB < NCHUNK)
                def _():
                    wait_scatter(q, b)

                    @pl.when(i + IDP < NCHUNK)
                    def _():
                        start_idx(i + IDP, q)

                    qn = (gg * NB + b + NB) % IDP
                    wait_idx(i + NB, qn)
                    start_gather(qn, b)

        return 0

    lax.fori_loop(0, NGRP // 2, group, 0)
    for b in range(NB):
        wait_scatter(NB + b, b)
    plsc.subcore_barrier()
    pltpu.sync_copy(aggr_sh.at[pl.ds(s * RPS, RPS)],
                    out_hbm.at[pl.ds(c * NPAD + s * RPS, RPS)])


@functools.cache
def _build_sc_deg():
    return functools.partial(
        pl.kernel,
        out_type=jax.ShapeDtypeStruct((2 * NPAD, EMB), jnp.float32),
        mesh=_mesh(),
        compiler_params=_SC_PARAMS,
        scratch_types=[
            pltpu.VMEM((IDP, CH), jnp.int32),
            pltpu.VMEM((CH, EMB), jnp.float32),
            pltpu.VMEM((ZR, EMB), jnp.float32),
            pltpu.VMEM_SHARED((NPAD, EMB), jnp.float32),
            [pltpu.SemaphoreType.DMA] * IDP,
            [pltpu.SemaphoreType.DMA] * NB,
        ],
    )(_sc_deg_body)


def _sc_deg_body(row3_hbm, out_hbm, ibuf, ones_v, zbuf, acc_sh,
                 isems, ssems):
    c = lax.axis_index("c")
    s = lax.axis_index("s")
    w = c * NS + s

    def start_idx(i, q):
        pltpu.async_copy(row3_hbm.at[w, i], ibuf.at[q], isems[q])

    def wait_idx(i, q):
        pltpu.make_async_copy(row3_hbm.at[w, i], ibuf.at[q], isems[q]).wait()

    def start_scatter(q, b):
        pltpu.async_copy(ones_v, acc_sh.at[ibuf.at[q]], ssems[b], add=True)

    def wait_scatter(q, b):
        pltpu.make_async_copy(ones_v, acc_sh.at[ibuf.at[q]], ssems[b]).wait()

    for q in range(IDP):
        start_idx(q, q)

    one = jnp.full((16,), 1.0, jnp.float32)

    def fill_ones(i, _):
        ones_v[i // 8, pl.ds((i % 8) * 16, 16)] = one
        return 0

    lax.fori_loop(0, CH * 8, fill_ones, 0)
    _zero_vmem(zbuf, ZR, EMB)
    for k in range(RPS // ZR):
        pltpu.sync_copy(zbuf, acc_sh.at[pl.ds(s * RPS + k * ZR, ZR)])
    plsc.subcore_barrier()

    def group(g2, _):
        for gg in range(2):
            for b in range(NB):
                i = g2 * IDP + gg * NB + b
                q = gg * NB + b
                qp = (q + NB) % IDP

                @pl.when(i >= NB)
                def _():
                    wait_scatter(qp, b)

                    @pl.when(i + NB < NCHUNK)
                    def _():
                        start_idx(i + NB, qp)

                wait_idx(i, q)
                start_scatter(q, b)

        return 0

    lax.fori_loop(0, NGRP // 2, group, 0)
    for b in range(NB):
        wait_scatter(NB + b, b)
    plsc.subcore_barrier()
    pltpu.sync_copy(acc_sh.at[pl.ds(s * RPS, RPS)],
                    out_hbm.at[pl.ds(c * NPAD + s * RPS, RPS)])


FCH = 2000


@functools.cache
def _build_sc_final():
    return functools.partial(
        pl.kernel,
        out_type=jax.ShapeDtypeStruct((E,), jnp.float32),
        mesh=_mesh(),
        compiler_params=_SC_PARAMS,
        scratch_types=[
            pltpu.VMEM((NPAD,), jnp.float32),
            pltpu.VMEM((NPAD,), jnp.float32),
            pltpu.VMEM((FCH,), jnp.int32),
            pltpu.VMEM((FCH,), jnp.int32),
            pltpu.VMEM((FCH,), jnp.float32),
        ],
    )(_sc_final_body)


def _sc_final_body(s1_hbm, s2_hbm, row_hbm, col_hbm, out_hbm,
                   s1_v, s2_v, row_v, col_v, out_v):
    c = lax.axis_index("c")
    s = lax.axis_index("s")
    base = c * (E // NC) + s * EPW

    pltpu.sync_copy(s1_hbm, s1_v)
    pltpu.sync_copy(s2_hbm, s2_v)

    def chunk(i, _):
        off = base + i * FCH
        pltpu.sync_copy(row_hbm.at[pl.ds(off, FCH)], row_v)
        pltpu.sync_copy(col_hbm.at[pl.ds(off, FCH)], col_v)

        def group(j, _):
            r = row_v[pl.ds(j * 16, 16)]
            cc = col_v[pl.ds(j * 16, 16)]
            v = plsc.load_gather(s1_v, [r]) + plsc.load_gather(s2_v, [cc])
            out_v[pl.ds(j * 16, 16)] = v
            return 0

        lax.fori_loop(0, FCH // 16, group, 0)
        pltpu.sync_copy(out_v, out_hbm.at[pl.ds(off, FCH)])
        return 0

    lax.fori_loop(0, EPW // FCH, chunk, 0)


BLK = 2560
G = NPAD // BLK


def _tc_first_body(x_ref, at_ref, d0_ref, d1_ref, w_ref, b_ref, bond_ref,
                   h_ref, dis_ref, dinv_ref, hx_ref, hxe2_ref):
    i = pl.program_id(0)
    rows = lax.broadcasted_iota(jnp.int32, (BLK, 1), 0) + i * BLK
    mask = (rows < N).astype(jnp.float32)
    oh = (x_ref[...] == lax.broadcasted_iota(jnp.int32, (1, 16), 1)
          ).astype(jnp.float32)
    h = jnp.dot(oh, at_ref[...], preferred_element_type=jnp.float32)
    h_ref[...] = h
    deg = d0_ref[0, :, 0:1] + d1_ref[0, :, 0:1] + 1.0
    dis = lax.rsqrt(deg) * mask
    dis_ref[...] = dis
    dinv_ref[...] = mask / deg
    hx = jnp.dot(h, w_ref[...], preferred_element_type=jnp.float32) + b_ref[...]
    hx_ref[...] = hx
    for a in range(4):
        hxe2_ref[a] = dis * jnp.maximum(hx + bond_ref[a:a + 1, :], 0.0)


_tc_first = pl.pallas_call(
    _tc_first_body,
    grid=(G,),
    in_specs=[
        pl.BlockSpec((BLK, 1), lambda i: (i, 0)),
        pl.BlockSpec((16, EMB), lambda i: (0, 0)),
        pl.BlockSpec((1, BLK, EMB), lambda i: (0, i, 0)),
        pl.BlockSpec((1, BLK, EMB), lambda i: (1, i, 0)),
        pl.BlockSpec((EMB, EMB), lambda i: (0, 0)),
        pl.BlockSpec((1, EMB), lambda i: (0, 0)),
        pl.BlockSpec((4, EMB), lambda i: (0, 0)),
    ],
    out_specs=[
        pl.BlockSpec((BLK, EMB), lambda i: (i, 0)),
        pl.BlockSpec((BLK, 1), lambda i: (i, 0)),
        pl.BlockSpec((BLK, 1), lambda i: (i, 0)),
        pl.BlockSpec((BLK, EMB), lambda i: (i, 0)),
        pl.BlockSpec((4, BLK, EMB), lambda i: (0, i, 0)),
    ],
    out_shape=(
        jax.ShapeDtypeStruct((NPAD, EMB), jnp.float32),
        jax.ShapeDtypeStruct((NPAD, 1), jnp.float32),
        jax.ShapeDtypeStruct((NPAD, 1), jnp.float32),
        jax.ShapeDtypeStruct((NPAD, EMB), jnp.float32),
        jax.ShapeDtypeStruct((4, NPAD, EMB), jnp.float32),
    ),
)


def _tc_comb_body(row_ref, attr_ref, comb_ref):
    comb_ref[...] = attr_ref[...] * NPAD + row_ref[...]


_tc_comb = pl.pallas_call(
    _tc_comb_body,
    out_shape=jax.ShapeDtypeStruct((E // EMB, EMB), jnp.int32),
)


def _tc_stats_body(p0_ref, p1_ref, hx_ref, dis_ref, dinv_ref, root_ref,
                   conv_ref, sums_ref, acc):
    i = pl.program_id(0)
    conv = (dis_ref[...] * (p0_ref[0] + p1_ref[0])
            + jnp.maximum(hx_ref[...] + root_ref[...], 0.0) * dinv_ref[...])
    conv_ref[...] = conv

    @pl.when(i == 0)
    def _():
        acc[...] = jnp.zeros((2, EMB), jnp.float32)

    acc[0:1, :] += jnp.sum(conv, axis=0, keepdims=True)
    acc[1:2, :] += jnp.sum(conv * conv, axis=0, keepdims=True)

    @pl.when(i == G - 1)
    def _():
        sums_ref[...] = acc[...]


_tc_stats = pl.pallas_call(
    _tc_stats_body,
    grid=(G,),
    in_specs=[
        pl.BlockSpec((1, BLK, EMB), lambda i: (0, i, 0)),
        pl.BlockSpec((1, BLK, EMB), lambda i: (1, i, 0)),
        pl.BlockSpec((BLK, EMB), lambda i: (i, 0)),
        pl.BlockSpec((BLK, 1), lambda i: (i, 0)),
        pl.BlockSpec((BLK, 1), lambda i: (i, 0)),
        pl.BlockSpec((1, EMB), lambda i: (0, 0)),
    ],
    out_specs=[
        pl.BlockSpec((BLK, EMB), lambda i: (i, 0)),
        pl.BlockSpec((2, EMB), lambda i: (0, 0)),
    ],
    out_shape=(
        jax.ShapeDtypeStruct((NPAD, EMB), jnp.float32),
        jax.ShapeDtypeStruct((2, EMB), jnp.float32),
    ),
    scratch_shapes=[pltpu.VMEM((2, EMB), jnp.float32)],
)


def _bn_next(conv_ref, h_ref, sums_ref, gamma_ref, beta_ref):
    mean = sums_ref[0:1, :] * (1.0 / N)
    var = sums_ref[1:2, :] * (1.0 / N) - mean * mean
    bn = ((conv_ref[...] - mean) * lax.rsqrt(var + 1e-5) * gamma_ref[...]
          + beta_ref[...])
    return jnp.maximum(bn, 0.0) + h_ref[...]


def _tc_apply_body(conv_ref, h_ref, sums_ref, gamma_ref, beta_ref,
                   w_ref, b_ref, bond_ref, dis_ref,
                   hn_ref, hx_ref, hxe2_ref):
    hn = _bn_next(conv_ref, h_ref, sums_ref, gamma_ref, beta_ref)
    hn_ref[...] = hn
    hx = jnp.dot(hn, w_ref[...], preferred_element_type=jnp.float32) + b_ref[...]
    hx_ref[...] = hx
    dis = dis_ref[...]
    for a in range(4):
        hxe2_ref[a] = dis * jnp.maximum(hx + bond_ref[a:a + 1, :], 0.0)


_tc_apply = pl.pallas_call(
    _tc_apply_body,
    grid=(G,),
    in_specs=[
        pl.BlockSpec((BLK, EMB), lambda i: (i, 0)),
        pl.BlockSpec((BLK, EMB), lambda i: (i, 0)),
        pl.BlockSpec((2, EMB), lambda i: (0, 0)),
        pl.BlockSpec((1, EMB), lambda i: (0, 0)),
        pl.BlockSpec((1, EMB), lambda i: (0, 0)),
        pl.BlockSpec((EMB, EMB), lambda i: (0, 0)),
        pl.BlockSpec((1, EMB), lambda i: (0, 0)),
        pl.BlockSpec((4, EMB), lambda i: (0, 0)),
        pl.BlockSpec((BLK, 1), lambda i: (i, 0)),
    ],
    out_specs=[
        pl.BlockSpec((BLK, EMB), lambda i: (i, 0)),
        pl.BlockSpec((BLK, EMB), lambda i: (i, 0)),
        pl.BlockSpec((4, BLK, EMB), lambda i: (0, i, 0)),
    ],
    out_shape=(
        jax.ShapeDtypeStruct((NPAD, EMB), jnp.float32),
        jax.ShapeDtypeStruct((NPAD, EMB), jnp.float32),
        jax.ShapeDtypeStruct((4, NPAD, EMB), jnp.float32),
    ),
)


def _tc_last_body(conv_ref, h_ref, sums_ref, gamma_ref, beta_ref,
                  w1_ref, w2_ref, be_ref, s1_ref, s2_ref):
    hn = _bn_next(conv_ref, h_ref, sums_ref, gamma_ref, beta_ref)
    s1_ref[...] = jnp.dot(hn, w1_ref[...],
                          preferred_element_type=jnp.float32) + be_ref[...]
    s2_ref[...] = jnp.dot(hn, w2_ref[...], preferred_element_type=jnp.float32)


_tc_last = pl.pallas_call(
    _tc_last_body,
    grid=(G,),
    in_specs=[
        pl.BlockSpec((BLK, EMB), lambda i: (i, 0)),
        pl.BlockSpec((BLK, EMB), lambda i: (i, 0)),
        pl.BlockSpec((2, EMB), lambda i: (0, 0)),
        pl.BlockSpec((1, EMB), lambda i: (0, 0)),
        pl.BlockSpec((1, EMB), lambda i: (0, 0)),
        pl.BlockSpec((EMB, 1), lambda i: (0, 0)),
        pl.BlockSpec((EMB, 1), lambda i: (0, 0)),
        pl.BlockSpec((1, 1), lambda i: (0, 0)),
    ],
    out_specs=[
        pl.BlockSpec((BLK, 1), lambda i: (i, 0)),
        pl.BlockSpec((BLK, 1), lambda i: (i, 0)),
    ],
    out_shape=(
        jax.ShapeDtypeStruct((NPAD, 1), jnp.float32),
        jax.ShapeDtypeStruct((NPAD, 1), jnp.float32),
    ),
)


def kernel(x, edge_index, edge_attr, atom_table, W_lin, b_lin, root_emb,
           bond_table, bn_gamma, bn_beta, W_ep, b_ep):
    xp = jnp.pad(x.astype(jnp.int32), (0, NPAD - N)).reshape(NPAD, 1)
    row = edge_index[0].astype(jnp.int32)
    col = edge_index[1].astype(jnp.int32)
    attr = edge_attr.astype(jnp.int32)
    row3 = row.reshape(NW, NCHUNK, CH)
    col3 = col.reshape(NW, NCHUNK, CH)

    degp = _build_sc_deg()(row3)
    degp_r = degp.reshape(2, NPAD, EMB)
    h, dis, dinv, hx, hxe2 = _tc_first(
        xp, atom_table, degp_r, degp_r, W_lin[0],
        b_lin[0].reshape(1, EMB), bond_table[0])
    comb2 = _tc_comb(row.reshape(E // EMB, EMB), attr.reshape(E // EMB, EMB))
    cc3 = jnp.stack([comb2.reshape(NW, NCHUNK, CH), col3], axis=2)

    for i in range(3):
        parts = _build_sc_edge()(hxe2.reshape(4 * NPAD, EMB), cc3)
        parts_r = parts.reshape(2, NPAD, EMB)
        conv, sums = _tc_stats(parts_r, parts_r, hx, dis, dinv, root_emb[i])
        if i < 2:
            h, hx, hxe2 = _tc_apply(
                conv, h, sums, bn_gamma[i].reshape(1, EMB),
                bn_beta[i].reshape(1, EMB), W_lin[i + 1],
                b_lin[i + 1].reshape(1, EMB), bond_table[i + 1], dis)
        else:
            s1, s2 = _tc_last(
                conv, h, sums, bn_gamma[i].reshape(1, EMB),
                bn_beta[i].reshape(1, EMB), W_ep[:EMB], W_ep[EMB:],
                b_ep.reshape(1, 1))

    out = _build_sc_final()(s1.reshape(NPAD), s2.reshape(NPAD), row, col)
    return out.reshape(E, 1)
